# baseline (speedup 1.0000x reference)
import functools

import jax
import jax.numpy as jnp
from jax import lax
from jax.experimental import pallas as pl
from jax.experimental.pallas import tpu as pltpu
from jax.experimental.pallas import tpu_sc as plsc

N_ROWS = 16384
N_COLS = 512
N_IDX = 128
LANES = 16

NUM_CORES = 2
NUM_SUBCORES = 16
NUM_WORKERS = NUM_CORES * NUM_SUBCORES
ROWS_PER_WORKER = N_ROWS // NUM_WORKERS
CHUNK_ROWS = 32
NUM_CHUNKS = ROWS_PER_WORKER // CHUNK_ROWS
IDX_GROUPS = N_IDX // LANES


def _body(in_hbm, idx_hbm, out_hbm, idx_v, in_buf, out_buf, sem_in, sem_out):
    wid = lax.axis_index("s") * NUM_CORES + lax.axis_index("c")
    row0 = wid * ROWS_PER_WORKER

    pltpu.sync_copy(idx_hbm, idx_v)
    idx_groups = [idx_v[pl.ds(g * LANES, LANES)] for g in range(IDX_GROUPS)]

    @pl.loop(0, NUM_CHUNKS)
    def _chunk(ch):
        base = row0 + ch * CHUNK_ROWS
        pltpu.async_copy(in_hbm.at[pl.ds(base, CHUNK_ROWS)], in_buf, sem_in).wait()

        @pl.loop(0, CHUNK_ROWS)
        def _row(r):
            row_idx = jnp.full((LANES,), r, dtype=jnp.int32)
            for g in range(IDX_GROUPS):
                vals = plsc.load_gather(in_buf, [row_idx, idx_groups[g]])
                out_buf[r, pl.ds(g * LANES, LANES)] = vals

        pltpu.async_copy(out_buf, out_hbm.at[pl.ds(base, CHUNK_ROWS)], sem_out).wait()


@jax.jit
def kernel(input, indexes):
    mesh = plsc.VectorSubcoreMesh(
        core_axis_name="c",
        subcore_axis_name="s",
        num_cores=NUM_CORES,
        num_subcores=NUM_SUBCORES,
    )
    run = pl.kernel(
        _body,
        out_type=jax.ShapeDtypeStruct((N_ROWS, N_IDX), jnp.float32),
        mesh=mesh,
        scratch_types=[
            pltpu.VMEM((N_IDX,), jnp.int32),
            pltpu.VMEM((CHUNK_ROWS, N_COLS), jnp.float32),
            pltpu.VMEM((CHUNK_ROWS, N_IDX), jnp.float32),
            pltpu.SemaphoreType.DMA,
            pltpu.SemaphoreType.DMA,
        ],
        compiler_params=pltpu.CompilerParams(
            use_tc_tiling_on_sc=False, needs_layout_passes=False
        ),
    )
    return run(input, indexes)


# speedup vs baseline: 1.0284x; 1.0284x over previous
import jax
import jax.numpy as jnp
from jax import lax
from jax.experimental import pallas as pl
from jax.experimental.pallas import tpu as pltpu
from jax.experimental.pallas import tpu_sc as plsc

N_ROWS = 16384
N_COLS = 512
N_IDX = 128
LANES = 16

NUM_CORES = 2
NUM_SUBCORES = 16
NUM_WORKERS = NUM_CORES * NUM_SUBCORES
ROWS_PER_WORKER = N_ROWS // NUM_WORKERS
CHUNK_ROWS = 64
NUM_CHUNKS = ROWS_PER_WORKER // CHUNK_ROWS
IDX_GROUPS = N_IDX // LANES
CHUNK_IN_WORDS = CHUNK_ROWS * N_COLS
CHUNK_OUT_WORDS = CHUNK_ROWS * N_IDX
GATHER_ITERS = CHUNK_OUT_WORDS // LANES


def _body(
    in_hbm,
    idx_hbm,
    out_hbm,
    idx_v,
    flat_idx,
    in_buf0,
    in_buf1,
    out_buf0,
    out_buf1,
    sem_i0,
    sem_i1,
    sem_o0,
    sem_o1,
):
    wid = lax.axis_index("s") * NUM_CORES + lax.axis_index("c")
    row0 = wid * ROWS_PER_WORKER

    in_bufs = (in_buf0, in_buf1)
    out_bufs = (out_buf0, out_buf1)
    sem_is = (sem_i0, sem_i1)
    sem_os = (sem_o0, sem_o1)

    pltpu.sync_copy(idx_hbm, idx_v)

    @pl.loop(0, CHUNK_ROWS)
    def _build(r):
        rbase = jnp.full((LANES,), r * N_COLS, dtype=jnp.int32)
        for g in range(IDX_GROUPS):
            flat_idx[pl.ds(r * N_IDX + g * LANES, LANES)] = (
                idx_v[pl.ds(g * LANES, LANES)] + rbase
            )

    def in_slice(ch):
        return in_hbm.at[pl.ds((row0 + ch * CHUNK_ROWS) * N_COLS, CHUNK_IN_WORDS)]

    def out_slice(ch):
        return out_hbm.at[pl.ds(row0 + ch * CHUNK_ROWS, CHUNK_ROWS)]

    pltpu.async_copy(in_slice(0), in_buf0, sem_i0)

    @pl.loop(0, NUM_CHUNKS, step=2)
    def _chunk(ch0):
        for b in range(2):
            ch = ch0 + b
            nb = 1 - b

            @pl.when(ch + 1 < NUM_CHUNKS)
            def _():
                pltpu.async_copy(in_slice(ch + 1), in_bufs[nb], sem_is[nb])

            pltpu.make_async_copy(in_slice(0), in_bufs[b], sem_is[b]).wait()

            @pl.when(ch >= 2)
            def _():
                pltpu.make_async_copy(out_bufs[b], out_slice(0), sem_os[b]).wait()

            @pl.loop(0, CHUNK_ROWS, unroll=2)
            def _gather(r):
                for g in range(IDX_GROUPS):
                    iv = flat_idx[pl.ds(r * N_IDX + g * LANES, LANES)]
                    out_bufs[b][r, pl.ds(g * LANES, LANES)] = plsc.load_gather(
                        in_bufs[b], [iv]
                    )

            pltpu.async_copy(out_bufs[b], out_slice(ch), sem_os[b])

    pltpu.make_async_copy(out_buf0, out_slice(0), sem_o0).wait()
    pltpu.make_async_copy(out_buf1, out_slice(0), sem_o1).wait()


@jax.jit
def kernel(input, indexes):
    mesh = plsc.VectorSubcoreMesh(
        core_axis_name="c",
        subcore_axis_name="s",
        num_cores=NUM_CORES,
        num_subcores=NUM_SUBCORES,
    )
    run = pl.kernel(
        _body,
        out_type=jax.ShapeDtypeStruct((N_ROWS, N_IDX), jnp.float32),
        mesh=mesh,
        scratch_types=[
            pltpu.VMEM((N_IDX,), jnp.int32),
            pltpu.VMEM((CHUNK_OUT_WORDS,), jnp.int32),
            pltpu.VMEM((CHUNK_IN_WORDS,), jnp.float32),
            pltpu.VMEM((CHUNK_IN_WORDS,), jnp.float32),
            pltpu.VMEM((CHUNK_ROWS, N_IDX), jnp.float32),
            pltpu.VMEM((CHUNK_ROWS, N_IDX), jnp.float32),
            pltpu.SemaphoreType.DMA,
            pltpu.SemaphoreType.DMA,
            pltpu.SemaphoreType.DMA,
            pltpu.SemaphoreType.DMA,
        ],
        compiler_params=pltpu.CompilerParams(
            use_tc_tiling_on_sc=False, needs_layout_passes=False
        ),
    )
    return run(input.reshape(-1), indexes)


# speedup vs baseline: 2.0970x; 2.0391x over previous
import jax
import jax.numpy as jnp
from jax import lax
from jax.experimental import pallas as pl
from jax.experimental.pallas import tpu as pltpu
from jax.experimental.pallas import tpu_sc as plsc

N_ROWS = 16384
N_COLS = 512
N_IDX = 128
LANES = 16

NUM_CORES = 2
NUM_SUBCORES = 16
NUM_WORKERS = NUM_CORES * NUM_SUBCORES
ROWS_PER_WORKER = N_ROWS // NUM_WORKERS
CHUNK_ROWS = 64
NUM_CHUNKS = ROWS_PER_WORKER // CHUNK_ROWS
IDX_GROUPS = N_IDX // LANES


def _body(
    in_hbm,
    idx_hbm,
    out_hbm,
    idx_v,
    in_buf0,
    in_buf1,
    out_buf0,
    out_buf1,
    sem_i0,
    sem_i1,
    sem_o0,
    sem_o1,
):
    wid = lax.axis_index("s") * NUM_CORES + lax.axis_index("c")
    row0 = wid * ROWS_PER_WORKER

    in_bufs = (in_buf0, in_buf1)
    out_bufs = (out_buf0, out_buf1)
    sem_is = (sem_i0, sem_i1)
    sem_os = (sem_o0, sem_o1)

    pltpu.sync_copy(idx_hbm, idx_v)
    idx_groups = [idx_v[pl.ds(g * LANES, LANES)] for g in range(IDX_GROUPS)]

    def in_slice(ch):
        return in_hbm.at[pl.ds(row0 + ch * CHUNK_ROWS, CHUNK_ROWS)]

    def out_slice(ch):
        return out_hbm.at[pl.ds(row0 + ch * CHUNK_ROWS, CHUNK_ROWS)]

    pltpu.async_copy(in_slice(0), in_buf0, sem_i0)

    @pl.loop(0, NUM_CHUNKS, step=2)
    def _chunk(ch0):
        for b in range(2):
            ch = ch0 + b
            nb = 1 - b

            @pl.when(ch + 1 < NUM_CHUNKS)
            def _():
                pltpu.async_copy(in_slice(ch + 1), in_bufs[nb], sem_is[nb])

            pltpu.make_async_copy(in_slice(0), in_bufs[b], sem_is[b]).wait()

            @pl.when(ch >= 2)
            def _():
                pltpu.make_async_copy(out_bufs[b], out_slice(0), sem_os[b]).wait()

            @pl.loop(0, CHUNK_ROWS, unroll=2)
            def _gather(r):
                rowv = jnp.full((LANES,), r, dtype=jnp.int32)
                for g in range(IDX_GROUPS):
                    out_bufs[b][r, pl.ds(g * LANES, LANES)] = plsc.load_gather(
                        in_bufs[b], [rowv, idx_groups[g]]
                    )

            pltpu.async_copy(out_bufs[b], out_slice(ch), sem_os[b])

    pltpu.make_async_copy(out_buf0, out_slice(0), sem_o0).wait()
    pltpu.make_async_copy(out_buf1, out_slice(0), sem_o1).wait()


@jax.jit
def kernel(input, indexes):
    mesh = plsc.VectorSubcoreMesh(
        core_axis_name="c",
        subcore_axis_name="s",
        num_cores=NUM_CORES,
        num_subcores=NUM_SUBCORES,
    )
    run = pl.kernel(
        _body,
        out_type=jax.ShapeDtypeStruct((N_ROWS, N_IDX), jnp.float32),
        mesh=mesh,
        scratch_types=[
            pltpu.VMEM((N_IDX,), jnp.int32),
            pltpu.VMEM((CHUNK_ROWS, N_COLS), jnp.float32),
            pltpu.VMEM((CHUNK_ROWS, N_COLS), jnp.float32),
            pltpu.VMEM((CHUNK_ROWS, N_IDX), jnp.float32),
            pltpu.VMEM((CHUNK_ROWS, N_IDX), jnp.float32),
            pltpu.SemaphoreType.DMA,
            pltpu.SemaphoreType.DMA,
            pltpu.SemaphoreType.DMA,
            pltpu.SemaphoreType.DMA,
        ],
        compiler_params=pltpu.CompilerParams(
            use_tc_tiling_on_sc=True, needs_layout_passes=False
        ),
    )
    return run(input, indexes)


# speedup vs baseline: 2.1004x; 1.0016x over previous
import jax
import jax.numpy as jnp
from jax import lax
from jax.experimental import pallas as pl
from jax.experimental.pallas import tpu as pltpu
from jax.experimental.pallas import tpu_sc as plsc

N_ROWS = 16384
N_COLS = 512
N_IDX = 128
LANES = 16

NUM_CORES = 2
NUM_SUBCORES = 16
NUM_WORKERS = NUM_CORES * NUM_SUBCORES
ROWS_PER_WORKER = N_ROWS // NUM_WORKERS
CHUNK_ROWS = 32
NUM_CHUNKS = ROWS_PER_WORKER // CHUNK_ROWS
IDX_GROUPS = N_IDX // LANES
NBUF = 4


def _body(in_hbm, idx_hbm, out_hbm, idx_v, in_bufs, out_bufs, sem_is, sem_os):
    wid = lax.axis_index("s") * NUM_CORES + lax.axis_index("c")
    row0 = wid * ROWS_PER_WORKER

    pltpu.sync_copy(idx_hbm, idx_v)
    idx_groups = [idx_v[pl.ds(g * LANES, LANES)] for g in range(IDX_GROUPS)]

    def in_slice(ch):
        return in_hbm.at[pl.ds(row0 + ch * CHUNK_ROWS, CHUNK_ROWS)]

    def out_slice(ch):
        return out_hbm.at[pl.ds(row0 + ch * CHUNK_ROWS, CHUNK_ROWS)]

    for ch in range(NBUF - 1):
        pltpu.async_copy(in_slice(ch), in_bufs[ch], sem_is[ch])

    @pl.loop(0, NUM_CHUNKS, step=NBUF)
    def _chunk(ch0):
        for b in range(NBUF):
            ch = ch0 + b

            @pl.when(ch + NBUF - 1 < NUM_CHUNKS)
            def _():
                nb = (b + NBUF - 1) % NBUF
                pltpu.async_copy(in_slice(ch + NBUF - 1), in_bufs[nb], sem_is[nb])

            pltpu.make_async_copy(in_slice(0), in_bufs[b], sem_is[b]).wait()

            @pl.when(ch >= NBUF)
            def _():
                pltpu.make_async_copy(out_bufs[b], out_slice(0), sem_os[b]).wait()

            @pl.loop(0, CHUNK_ROWS, unroll=2)
            def _gather(r):
                rowv = jnp.full((LANES,), r, dtype=jnp.int32)
                for g in range(IDX_GROUPS):
                    out_bufs[b][r, pl.ds(g * LANES, LANES)] = plsc.load_gather(
                        in_bufs[b], [rowv, idx_groups[g]]
                    )

            pltpu.async_copy(out_bufs[b], out_slice(ch), sem_os[b])

    for b in range(NBUF):
        pltpu.make_async_copy(out_bufs[b], out_slice(0), sem_os[b]).wait()


@jax.jit
def kernel(input, indexes):
    mesh = plsc.VectorSubcoreMesh(
        core_axis_name="c",
        subcore_axis_name="s",
        num_cores=NUM_CORES,
        num_subcores=NUM_SUBCORES,
    )
    run = pl.kernel(
        _body,
        out_type=jax.ShapeDtypeStruct((N_ROWS, N_IDX), jnp.float32),
        mesh=mesh,
        scratch_types=[
            pltpu.VMEM((N_IDX,), jnp.int32),
            [pltpu.VMEM((CHUNK_ROWS, N_COLS), jnp.float32) for _ in range(NBUF)],
            [pltpu.VMEM((CHUNK_ROWS, N_IDX), jnp.float32) for _ in range(NBUF)],
            [pltpu.SemaphoreType.DMA for _ in range(NBUF)],
            [pltpu.SemaphoreType.DMA for _ in range(NBUF)],
        ],
        compiler_params=pltpu.CompilerParams(
            use_tc_tiling_on_sc=True, needs_layout_passes=False
        ),
    )
    return run(input, indexes)
